# baseline (speedup 1.0000x reference)
import functools

import jax
import jax.numpy as jnp
from jax import lax
from jax.experimental import pallas as pl
from jax.experimental.pallas import tpu as pltpu
from jax.experimental.pallas import tpu_sc as plsc

H = 768
NH = 12
HD = 64
NE = 8
L = 2048
BLK = 128
NBLKS = L // BLK + NE
P = NBLKS * BLK


def _shift_down(a, k):
    return jnp.concatenate(
        [jnp.zeros((k, a.shape[1]), a.dtype), a[:-k]], axis=0)


def _shift_right(a, k):
    return jnp.concatenate(
        [jnp.zeros((a.shape[0], k), a.dtype), a[:, :-k]], axis=1)


def _route_body(y_ref, g_ref, b_ref, rw_ref, rb_ref,
                x1_ref, pos_ref, gid_ref, val_ref):
    y = y_ref[...]
    mu = jnp.mean(y, axis=1, keepdims=True)
    var = jnp.mean((y - mu) ** 2, axis=1, keepdims=True)
    x1 = (y - mu) / jnp.sqrt(var + 1e-5) * g_ref[...] + b_ref[...]
    x1_ref[...] = x1

    logits = (
        jnp.dot(x1.astype(jnp.bfloat16), rw_ref[...],
                preferred_element_type=jnp.float32)
        + rb_ref[...]
    )
    lanes = lax.broadcasted_iota(jnp.int32, (L, 128), 1)
    mx1 = jnp.max(logits, axis=1, keepdims=True)
    i1 = jnp.min(jnp.where(logits == mx1, lanes, 127), axis=1, keepdims=True)
    l2 = jnp.where(lanes == i1, -jnp.inf, logits)
    mx2 = jnp.max(l2, axis=1, keepdims=True)
    i2 = jnp.min(jnp.where(l2 == mx2, lanes, 127), axis=1, keepdims=True)
    e = jnp.maximum(i1, i2)

    oh = (lanes == e).astype(jnp.int32)
    cs = oh
    k = 1
    while k < L:
        cs = cs + _shift_down(cs, k)
        k *= 2
    counts = cs[L - 1:L, :]
    nblk = (counts + (BLK - 1)) // BLK
    cnb = nblk
    k = 1
    while k < 128:
        cnb = cnb + _shift_right(cnb, k)
        k *= 2
    padoff = (cnb - nblk) * BLK
    pos = jnp.sum(oh * (padoff + cs - oh), axis=1, keepdims=True)
    pos_ref[...] = jnp.broadcast_to(pos, (L, 128))

    biota = lax.broadcasted_iota(jnp.int32, (1, 128), 1)
    gid = jnp.zeros((1, 128), jnp.int32)
    for ei in range(NE):
        gid = gid + (biota >= cnb[0:1, ei:ei + 1]).astype(jnp.int32)
    gid_ref[...] = jnp.minimum(gid, NE - 1)
    val_ref[...] = (biota < cnb[0:1, NE - 1:NE]).astype(jnp.int32)


def _route(y, g, b, rw_pad, rb_pad):
    return pl.pallas_call(
        _route_body,
        grid=(1,),
        in_specs=[
            pl.BlockSpec((L, H), lambda i: (0, 0)),
            pl.BlockSpec((1, H), lambda i: (0, 0)),
            pl.BlockSpec((1, H), lambda i: (0, 0)),
            pl.BlockSpec((H, 128), lambda i: (0, 0)),
            pl.BlockSpec((1, 128), lambda i: (0, 0)),
        ],
        out_specs=[
            pl.BlockSpec((L, H), lambda i: (0, 0)),
            pl.BlockSpec((L, 128), lambda i: (0, 0)),
            pl.BlockSpec((1, 128), lambda i: (0, 0)),
            pl.BlockSpec((1, 128), lambda i: (0, 0)),
        ],
        out_shape=[
            jax.ShapeDtypeStruct((L, H), jnp.float32),
            jax.ShapeDtypeStruct((L, 128), jnp.int32),
            jax.ShapeDtypeStruct((1, 128), jnp.int32),
            jax.ShapeDtypeStruct((1, 128), jnp.int32),
        ],
    )(y, g, b, rw_pad, rb_pad)


_SC_NC = 2
_SC_NS = 16
_NW = _SC_NC * _SC_NS
_TPW = L // _NW


@functools.cache
def _sc_kernels():
    mesh = plsc.VectorSubcoreMesh(core_axis_name="c", subcore_axis_name="s")

    def wid():
        return lax.axis_index("s") * _SC_NC + lax.axis_index("c")

    scratch = [
        pltpu.VMEM((_TPW,), jnp.int32),
        pltpu.VMEM((_TPW, H), jnp.float32),
        pltpu.SemaphoreType.DMA,
    ]

    @functools.partial(
        pl.kernel,
        out_type=jax.ShapeDtypeStruct((P, H), jnp.float32),
        mesh=mesh, scratch_types=scratch)
    def scatter_k(x1_hbm, pos_hbm, xs_hbm, idx_v, rows_v, sem):
        base = wid() * _TPW
        pltpu.sync_copy(pos_hbm.at[pl.ds(base, _TPW)], idx_v)
        pltpu.sync_copy(x1_hbm.at[pl.ds(base, _TPW)], rows_v)
        pltpu.async_copy(rows_v, xs_hbm.at[idx_v], sem).wait()

    @functools.partial(
        pl.kernel,
        out_type=jax.ShapeDtypeStruct((L, H), jnp.float32),
        mesh=mesh, scratch_types=scratch)
    def gather_k(outs_hbm, pos_hbm, out_hbm, idx_v, rows_v, sem):
        base = wid() * _TPW
        pltpu.sync_copy(pos_hbm.at[pl.ds(base, _TPW)], idx_v)
        pltpu.async_copy(outs_hbm.at[idx_v], rows_v, sem).wait()
        pltpu.sync_copy(rows_v, out_hbm.at[pl.ds(base, _TPW)])

    return scatter_k, gather_k


def _dispatch_scatter(x1, pos):
    return _sc_kernels()[0](x1, pos)


def _return_gather(outs, pos):
    return _sc_kernels()[1](outs, pos)


def _ffn_body(gid_ref, val_ref, xs_ref, w1_ref, b1_ref, w2_ref, b2_ref,
              g_ref, b_ref, o_ref):
    bidx = pl.program_id(0)

    @pl.when(val_ref[bidx] == 1)
    def _():
        xb = xs_ref[...]
        h1 = lax.dot_general(xb.astype(jnp.bfloat16), w1_ref[0],
                             (((1,), (1,)), ((), ())),
                             preferred_element_type=jnp.float32) + b1_ref[0]
        h1 = 0.5 * h1 * (1.0 + lax.erf(h1 * 0.7071067811865476))
        h2 = lax.dot_general(h1.astype(jnp.bfloat16), w2_ref[0],
                             (((1,), (1,)), ((), ())),
                             preferred_element_type=jnp.float32) + b2_ref[0]
        y = xb + h2
        mu = jnp.mean(y, axis=1, keepdims=True)
        var = jnp.mean((y - mu) ** 2, axis=1, keepdims=True)
        o_ref[...] = (y - mu) / jnp.sqrt(var + 1e-5) * g_ref[...] + b_ref[...]


def _ffn(gids, valid, xs, w1, b1, w2, b2, g, b):
    grid_spec = pltpu.PrefetchScalarGridSpec(
        num_scalar_prefetch=2,
        grid=(NBLKS,),
        in_specs=[
            pl.BlockSpec((BLK, H), lambda i, gr, vr: (i, 0)),
            pl.BlockSpec((1, H, H), lambda i, gr, vr: (gr[i], 0, 0)),
            pl.BlockSpec((1, 1, H), lambda i, gr, vr: (gr[i], 0, 0)),
            pl.BlockSpec((1, H, H), lambda i, gr, vr: (gr[i], 0, 0)),
            pl.BlockSpec((1, 1, H), lambda i, gr, vr: (gr[i], 0, 0)),
            pl.BlockSpec((1, H), lambda i, gr, vr: (0, 0)),
            pl.BlockSpec((1, H), lambda i, gr, vr: (0, 0)),
        ],
        out_specs=pl.BlockSpec((BLK, H), lambda i, gr, vr: (i, 0)),
    )
    return pl.pallas_call(
        _ffn_body,
        grid_spec=grid_spec,
        out_shape=jax.ShapeDtypeStruct((P, H), jnp.float32),
    )(gids, valid, xs, w1, b1, w2, b2, g, b)


def kernel(x, in_proj_w, in_proj_b, out_proj_w, out_proj_b, ln1_g, ln1_b,
           router_w, router_b, w1, b1, w2, b2, ln2_g, ln2_b):
    x2 = x[:, 0, :]

    qkv = x2 @ in_proj_w.T + in_proj_b
    q, k, v = jnp.split(qkv, 3, axis=-1)
    rs = lambda t: t.reshape(L, NH, HD).transpose(1, 0, 2)
    q = rs(q) / (HD ** 0.5)
    k = rs(k)
    v = rs(v)
    scores = jnp.einsum('bld,bmd->blm', q, k)
    attn = jax.nn.softmax(scores, axis=-1)
    out = jnp.einsum('blm,bmd->bld', attn, v)
    out = out.transpose(1, 0, 2).reshape(L, H)
    a = out @ out_proj_w.T + out_proj_b
    y = x2 + a

    rw_pad = jnp.zeros((H, 128), jnp.float32).at[:, :NE].set(
        router_w.T).astype(jnp.bfloat16)
    rb_pad = jnp.full((128,), -1e30, jnp.float32).at[:NE].set(
        router_b).reshape(1, 128)
    x1, pos_b, gid_b, val_b = _route(y, ln1_g.reshape(1, H),
                                     ln1_b.reshape(1, H), rw_pad, rb_pad)
    pos = pos_b[:, 0]
    gids = gid_b[0, :NBLKS]
    valid = val_b[0, :NBLKS]

    xs = _dispatch_scatter(x1, pos)
    outs = _ffn(gids, valid, xs, w1.astype(jnp.bfloat16),
                b1.reshape(NE, 1, H), w2.astype(jnp.bfloat16),
                b2.reshape(NE, 1, H),
                ln2_g.reshape(1, H), ln2_b.reshape(1, H))
    out_rows = _return_gather(outs, pos)
    return out_rows.reshape(L, 1, H)


# speedup vs baseline: 1.0399x; 1.0399x over previous
import functools

import jax
import jax.numpy as jnp
from jax import lax
from jax.experimental import pallas as pl
from jax.experimental.pallas import tpu as pltpu
from jax.experimental.pallas import tpu_sc as plsc

H = 768
NH = 12
HD = 64
NE = 8
L = 2048
BLK = 128
NBLKS = L // BLK + NE
P = NBLKS * BLK


def _shift_down(a, k):
    return jnp.concatenate(
        [jnp.zeros((k, a.shape[1]), a.dtype), a[:-k]], axis=0)


def _shift_right(a, k):
    return jnp.concatenate(
        [jnp.zeros((a.shape[0], k), a.dtype), a[:, :-k]], axis=1)


def _route_body(y_ref, g_ref, b_ref, rw_ref, rb_ref,
                x1_ref, pos_ref, gid_ref, val_ref):
    y = y_ref[...]
    mu = jnp.mean(y, axis=1, keepdims=True)
    var = jnp.mean((y - mu) ** 2, axis=1, keepdims=True)
    x1 = (y - mu) / jnp.sqrt(var + 1e-5) * g_ref[...] + b_ref[...]
    x1_ref[...] = x1

    logits = (
        jnp.dot(x1, rw_ref[...], preferred_element_type=jnp.float32)
        + rb_ref[...]
    )
    lanes = lax.broadcasted_iota(jnp.int32, (L, 128), 1)
    mx1 = jnp.max(logits, axis=1, keepdims=True)
    i1 = jnp.min(jnp.where(logits == mx1, lanes, 127), axis=1, keepdims=True)
    l2 = jnp.where(lanes == i1, -jnp.inf, logits)
    mx2 = jnp.max(l2, axis=1, keepdims=True)
    i2 = jnp.min(jnp.where(l2 == mx2, lanes, 127), axis=1, keepdims=True)
    e = jnp.maximum(i1, i2)

    oh = (lanes == e).astype(jnp.int32)
    cs = oh
    k = 1
    while k < L:
        cs = cs + _shift_down(cs, k)
        k *= 2
    counts = cs[L - 1:L, :]
    nblk = (counts + (BLK - 1)) // BLK
    cnb = nblk
    k = 1
    while k < 128:
        cnb = cnb + _shift_right(cnb, k)
        k *= 2
    padoff = (cnb - nblk) * BLK
    pos = jnp.sum(oh * (padoff + cs - oh), axis=1, keepdims=True)
    pos_ref[...] = jnp.broadcast_to(pos, (L, 128))

    biota = lax.broadcasted_iota(jnp.int32, (1, 128), 1)
    gid = jnp.zeros((1, 128), jnp.int32)
    for ei in range(NE):
        gid = gid + (biota >= cnb[0:1, ei:ei + 1]).astype(jnp.int32)
    gid_ref[...] = jnp.minimum(gid, NE - 1)
    val_ref[...] = (biota < cnb[0:1, NE - 1:NE]).astype(jnp.int32)


def _route(y, g, b, rw_pad, rb_pad):
    return pl.pallas_call(
        _route_body,
        grid=(1,),
        in_specs=[
            pl.BlockSpec((L, H), lambda i: (0, 0)),
            pl.BlockSpec((1, H), lambda i: (0, 0)),
            pl.BlockSpec((1, H), lambda i: (0, 0)),
            pl.BlockSpec((H, 128), lambda i: (0, 0)),
            pl.BlockSpec((1, 128), lambda i: (0, 0)),
        ],
        out_specs=[
            pl.BlockSpec((L, H), lambda i: (0, 0)),
            pl.BlockSpec((L, 128), lambda i: (0, 0)),
            pl.BlockSpec((1, 128), lambda i: (0, 0)),
            pl.BlockSpec((1, 128), lambda i: (0, 0)),
        ],
        out_shape=[
            jax.ShapeDtypeStruct((L, H), jnp.float32),
            jax.ShapeDtypeStruct((L, 128), jnp.int32),
            jax.ShapeDtypeStruct((1, 128), jnp.int32),
            jax.ShapeDtypeStruct((1, 128), jnp.int32),
        ],
    )(y, g, b, rw_pad, rb_pad)


_SC_NC = 2
_SC_NS = 16
_NW = _SC_NC * _SC_NS
_TPW = L // _NW


@functools.cache
def _sc_kernels():
    mesh = plsc.VectorSubcoreMesh(core_axis_name="c", subcore_axis_name="s")

    def wid():
        return lax.axis_index("s") * _SC_NC + lax.axis_index("c")

    scratch = [
        pltpu.VMEM((_TPW,), jnp.int32),
        pltpu.VMEM((_TPW, H), jnp.float32),
        pltpu.SemaphoreType.DMA,
    ]

    @functools.partial(
        pl.kernel,
        out_type=jax.ShapeDtypeStruct((P, H), jnp.float32),
        mesh=mesh, scratch_types=scratch)
    def scatter_k(x1_hbm, pos_hbm, xs_hbm, idx_v, rows_v, sem):
        base = wid() * _TPW
        pltpu.sync_copy(pos_hbm.at[pl.ds(base, _TPW)], idx_v)
        pltpu.sync_copy(x1_hbm.at[pl.ds(base, _TPW)], rows_v)
        pltpu.async_copy(rows_v, xs_hbm.at[idx_v], sem).wait()

    @functools.partial(
        pl.kernel,
        out_type=jax.ShapeDtypeStruct((L, H), jnp.float32),
        mesh=mesh, scratch_types=scratch)
    def gather_k(outs_hbm, pos_hbm, out_hbm, idx_v, rows_v, sem):
        base = wid() * _TPW
        pltpu.sync_copy(pos_hbm.at[pl.ds(base, _TPW)], idx_v)
        pltpu.async_copy(outs_hbm.at[idx_v], rows_v, sem).wait()
        pltpu.sync_copy(rows_v, out_hbm.at[pl.ds(base, _TPW)])

    return scatter_k, gather_k


def _dispatch_scatter(x1, pos):
    return _sc_kernels()[0](x1, pos)


def _return_gather(outs, pos):
    return _sc_kernels()[1](outs, pos)


def _ffn_body(gid_ref, val_ref, xs_ref, w1_ref, b1_ref, w2_ref, b2_ref,
              g_ref, b_ref, o_ref):
    bidx = pl.program_id(0)

    @pl.when(val_ref[bidx] == 1)
    def _():
        xb = xs_ref[...]
        h1 = lax.dot_general(xb, w1_ref[0], (((1,), (1,)), ((), ())),
                             preferred_element_type=jnp.float32) + b1_ref[0]
        h1 = 0.5 * h1 * (1.0 + lax.erf(h1 * 0.7071067811865476))
        h2 = lax.dot_general(h1, w2_ref[0], (((1,), (1,)), ((), ())),
                             preferred_element_type=jnp.float32) + b2_ref[0]
        y = xb + h2
        mu = jnp.mean(y, axis=1, keepdims=True)
        var = jnp.mean((y - mu) ** 2, axis=1, keepdims=True)
        o_ref[...] = (y - mu) / jnp.sqrt(var + 1e-5) * g_ref[...] + b_ref[...]


def _ffn(gids, valid, xs, w1, b1, w2, b2, g, b):
    grid_spec = pltpu.PrefetchScalarGridSpec(
        num_scalar_prefetch=2,
        grid=(NBLKS,),
        in_specs=[
            pl.BlockSpec((BLK, H), lambda i, gr, vr: (i, 0)),
            pl.BlockSpec((1, H, H), lambda i, gr, vr: (gr[i], 0, 0)),
            pl.BlockSpec((1, 1, H), lambda i, gr, vr: (gr[i], 0, 0)),
            pl.BlockSpec((1, H, H), lambda i, gr, vr: (gr[i], 0, 0)),
            pl.BlockSpec((1, 1, H), lambda i, gr, vr: (gr[i], 0, 0)),
            pl.BlockSpec((1, H), lambda i, gr, vr: (0, 0)),
            pl.BlockSpec((1, H), lambda i, gr, vr: (0, 0)),
        ],
        out_specs=pl.BlockSpec((BLK, H), lambda i, gr, vr: (i, 0)),
    )
    return pl.pallas_call(
        _ffn_body,
        grid_spec=grid_spec,
        out_shape=jax.ShapeDtypeStruct((P, H), jnp.float32),
    )(gids, valid, xs, w1, b1, w2, b2, g, b)


def kernel(x, in_proj_w, in_proj_b, out_proj_w, out_proj_b, ln1_g, ln1_b,
           router_w, router_b, w1, b1, w2, b2, ln2_g, ln2_b):
    x2 = x[:, 0, :]

    qkv = x2 @ in_proj_w.T + in_proj_b
    q, k, v = jnp.split(qkv, 3, axis=-1)
    rs = lambda t: t.reshape(L, NH, HD).transpose(1, 0, 2)
    q = rs(q) / (HD ** 0.5)
    k = rs(k)
    v = rs(v)
    scores = jnp.einsum('bld,bmd->blm', q, k)
    attn = jax.nn.softmax(scores, axis=-1)
    out = jnp.einsum('blm,bmd->bld', attn, v)
    out = out.transpose(1, 0, 2).reshape(L, H)
    a = out @ out_proj_w.T + out_proj_b
    y = x2 + a

    rw_pad = jnp.zeros((H, 128), jnp.float32).at[:, :NE].set(router_w.T)
    rb_pad = jnp.full((128,), -1e30, jnp.float32).at[:NE].set(
        router_b).reshape(1, 128)
    x1, pos_b, gid_b, val_b = _route(y, ln1_g.reshape(1, H),
                                     ln1_b.reshape(1, H), rw_pad, rb_pad)
    pos = pos_b[:, 0]
    gids = gid_b[0, :NBLKS]
    valid = val_b[0, :NBLKS]

    xs = _dispatch_scatter(x1, pos)
    outs = _ffn(gids, valid, xs, w1, b1.reshape(NE, 1, H),
                w2, b2.reshape(NE, 1, H),
                ln2_g.reshape(1, H), ln2_b.reshape(1, H))
    out_rows = _return_gather(outs, pos)
    return out_rows.reshape(L, 1, H)
